# baseline (speedup 1.0000x reference)
import functools

import jax
import jax.numpy as jnp
import numpy as np
from jax.experimental import pallas as pl

N, D, C0, F0 = 10000, 16, 1, 128
C1, F1 = 4, 16
C2, F2 = 32, 1
NUM_CLASS = 40

BLOCK_N = 40


def _softsign(v):
    return v / (1.0 + jnp.abs(v))


def _klgl_block(x_ref, nbr_ref, w1_ref, b1_ref, w2_ref, b2_ref, wc_ref,
                bc_ref, out_ref):
    B = x_ref.shape[0]
    xb = x_ref[:]
    nb = nbr_ref[:]
    s = jnp.sum(nb, axis=1)

    a = xb[:, :, None] * s[:, None, :] + s[:, :, None] * xb[:, None, :]
    m = jnp.sign(a) * jnp.sqrt(jnp.abs(a))
    rs = jnp.sum(jnp.abs(m), axis=2) + 1e-7

    v = jnp.concatenate([xb[:, None, :], nb], axis=1)
    z = jax.lax.dot_general(v, m, (((2,), (1,)), ((0,), (0,))),
                            preferred_element_type=jnp.float32)
    z = z / rs[:, None, :]

    h = jnp.dot(z.reshape(B * 17, F0), w1_ref[:],
                preferred_element_type=jnp.float32) + b1_ref[:]
    h = _softsign(h).reshape(B, 17, C1 * F1)

    x1 = h[:, 0, :].reshape(B, C1, F1)
    s2 = jnp.sum(h[:, 1:, :], axis=1).reshape(B, C1, F1)

    a2 = (x1[:, :, :, None] * s2[:, :, None, :]
          + s2[:, :, :, None] * x1[:, :, None, :])
    r2 = jnp.sqrt(jnp.abs(a2))
    m2 = jnp.sign(a2) * r2
    rs2 = jnp.sum(r2, axis=3) + 1e-7
    zx2 = jnp.sum(m2 * x1[:, :, :, None], axis=2) / rs2

    x2 = jnp.dot(zx2.reshape(B, C1 * F1), w2_ref[:],
                 preferred_element_type=jnp.float32) + b2_ref[:]
    x2 = _softsign(x2)
    out_ref[:] = jnp.dot(x2, wc_ref[:],
                         preferred_element_type=jnp.float32) + bc_ref[:]


@jax.jit
def kernel(x, neighbor, W1, b1, g1, be1, W2, b2, g2, be2, Wc, bc):
    inv = 1.0 / np.sqrt(1.0 + 1e-5)
    s1 = inv * jnp.repeat(g1, F1)
    w1f = W1.reshape(C1 * F1, C0 * F0).T * s1[None, :]
    b1f = (b1 * s1 + jnp.repeat(be1, F1))[None, :]
    s2 = inv * jnp.repeat(g2, F2)
    w2f = W2.reshape(C2 * F2, C1 * F1).T * s2[None, :]
    b2f = (b2 * s2 + jnp.repeat(be2, F2))[None, :]
    wct = Wc.T
    bcf = bc[None, :]

    xr = x.reshape(N, F0)
    nr = neighbor.reshape(N, D, F0)

    grid = (N // BLOCK_N,)
    out = pl.pallas_call(
        _klgl_block,
        grid=grid,
        in_specs=[
            pl.BlockSpec((BLOCK_N, F0), lambda i: (i, 0)),
            pl.BlockSpec((BLOCK_N, D, F0), lambda i: (i, 0, 0)),
            pl.BlockSpec((C0 * F0, C1 * F1), lambda i: (0, 0)),
            pl.BlockSpec((1, C1 * F1), lambda i: (0, 0)),
            pl.BlockSpec((C1 * F1, C2 * F2), lambda i: (0, 0)),
            pl.BlockSpec((1, C2 * F2), lambda i: (0, 0)),
            pl.BlockSpec((C2 * F2, NUM_CLASS), lambda i: (0, 0)),
            pl.BlockSpec((1, NUM_CLASS), lambda i: (0, 0)),
        ],
        out_specs=pl.BlockSpec((BLOCK_N, NUM_CLASS), lambda i: (i, 0)),
        out_shape=jax.ShapeDtypeStruct((N, NUM_CLASS), jnp.float32),
    )(xr, nr, w1f, b1f, w2f, b2f, wct, bcf)
    return out
